# baseline (speedup 1.0000x reference)
import jax
import jax.numpy as jnp
from jax import lax
from jax.experimental import pallas as pl
from jax.experimental.pallas import tpu as pltpu
from jax.experimental.pallas import tpu_sc as plsc

BATCH = 4096
BLOCK = 200
EMB = 64
PEMB = 128
NC = 2
NS = 16
NW = NC * NS
BB = BATCH // NW
PB = 8
NCHUNK = BLOCK // PB
TPITCH = 131
EBK = EMB // 8


def _body(idx_hbm, tok_hbm, pos_hbm, out_hbm, idx_v, rows_v, pos_v, t_v,
          isem, gsem, osem):
    wid = lax.axis_index("s") * NC + lax.axis_index("c")
    pltpu.sync_copy(pos_hbm, pos_v)
    lane = lax.iota(jnp.int32, 16)
    evecs = [16 * j + lane for j in range(EMB // 16)]

    def fire_idx(pb):
        s = lax.rem(pb, 2)
        pltpu.async_copy(idx_hbm.at[pb, wid], idx_v.at[s], isem.at[s])

    def wait_idx(pb):
        s = lax.rem(pb, 2)
        pltpu.make_async_copy(idx_hbm.at[0, wid], idx_v.at[s], isem.at[s]).wait()

    def fire_g(p):
        s = lax.rem(p, 2)
        si = lax.rem(lax.div(p, PB), 2)
        pltpu.async_copy(
            tok_hbm.at[idx_v.at[si, lax.rem(p, PB)]],
            rows_v.at[pl.ds(s * BB, BB)],
            gsem.at[s],
        )

    def wait_g(p):
        s = lax.rem(p, 2)
        pltpu.make_async_copy(
            tok_hbm.at[pl.ds(0, BB)], rows_v.at[pl.ds(0, BB)], gsem.at[s]
        ).wait()

    def wait_store(p):
        s = lax.rem(p, 2)
        for _ in range(EBK):
            pltpu.make_async_copy(
                t_v.at[s, pl.ds(0, 8), pl.ds(0, BB)],
                out_hbm.at[0, 0, wid],
                osem.at[s],
            ).wait()

    def tq(p, skip_wait=False):
        s = lax.rem(p, 2)
        if not skip_wait:
            wait_store(p - 2)
        tref = t_v.at[s]
        pvs = [pos_v[p, pl.ds(16 * j, 16)] for j in range(EMB // 16)]

        def rbody(r, carry):
            bc = jnp.full((16,), r, dtype=jnp.int32)
            row = s * BB + r
            for j in range(EMB // 16):
                vals = rows_v[row, pl.ds(16 * j, 16)] + pvs[j]
                plsc.store_scatter(tref, [evecs[j], bc], vals)
            return carry

        lax.fori_loop(0, BB, rbody, 0)
        for ebk in range(EBK):
            pltpu.async_copy(
                t_v.at[s, pl.ds(ebk * 8, 8), pl.ds(0, BB)],
                out_hbm.at[p, ebk, wid],
                osem.at[s],
            )

    fire_idx(0)
    fire_idx(1)
    wait_idx(0)
    fire_g(0)
    fire_g(1)
    wait_g(0)

    def chunk_body(pb, carry, first=False, last_idx=False):
        p0 = 8 * pb
        tq(p0, skip_wait=first)
        wait_idx(pb + 1)
        fire_g(p0 + 2)
        wait_g(p0 + 1)
        tq(p0 + 1, skip_wait=first)
        for i in range(2, 7):
            fire_g(p0 + i + 1)
            wait_g(p0 + i)
            tq(p0 + i)
        fire_g(p0 + 8)
        wait_g(p0 + 7)
        if not last_idx:
            fire_idx(pb + 2)
        tq(p0 + 7)
        fire_g(p0 + 9)
        wait_g(p0 + 8)
        return carry

    chunk_body(0, 0, first=True)
    lax.fori_loop(1, NCHUNK - 2, chunk_body, 0)
    chunk_body(NCHUNK - 2, 0, last_idx=True)
    p0 = 192
    for i in range(6):
        tq(p0 + i)
        fire_g(p0 + i + 2)
        wait_g(p0 + i + 1)
    tq(p0 + 6)
    wait_g(199)
    tq(199)
    wait_store(198)
    wait_store(199)


@jax.jit
def kernel(input, tok_table, pos_table):
    idx_r = input.T.reshape(NCHUNK, PB, NW, BB).transpose(0, 2, 1, 3)
    tok_pad = jnp.pad(tok_table, ((0, 0), (0, PEMB - EMB)))
    pos_pad = jnp.pad(pos_table, ((0, 0), (0, PEMB - EMB)))
    mesh = plsc.VectorSubcoreMesh(core_axis_name="c", subcore_axis_name="s")
    out5d = pl.kernel(
        _body,
        out_type=jax.ShapeDtypeStruct((BLOCK, EBK, NW, 8, BB), jnp.float32),
        mesh=mesh,
        scratch_types=[
            pltpu.VMEM((2, PB, BB), jnp.int32),
            pltpu.VMEM((2 * BB, PEMB), jnp.float32),
            pltpu.VMEM((BLOCK, PEMB), jnp.float32),
            pltpu.VMEM((2, EMB, TPITCH), jnp.float32),
            pltpu.SemaphoreType.DMA((2,)),
            pltpu.SemaphoreType.DMA((2,)),
            pltpu.SemaphoreType.DMA((2,)),
        ],
        compiler_params=pltpu.CompilerParams(
            use_tc_tiling_on_sc=True, needs_layout_passes=False
        ),
    )(idx_r, tok_pad, pos_pad)
    out = out5d.transpose(2, 4, 0, 1, 3).reshape(BATCH, BLOCK, EMB)
    return out


# speedup vs baseline: 2.2010x; 2.2010x over previous
import jax
import jax.numpy as jnp
from jax import lax
from jax.experimental import pallas as pl
from jax.experimental.pallas import tpu as pltpu
from jax.experimental.pallas import tpu_sc as plsc

BATCH = 4096
BLOCK = 200
EMB = 64
NC = 2
NS = 16
NW = NC * NS
BB = BATCH // NW
PB = 8
NCHUNK = BLOCK // PB
HP = 4
HROWS = HP * BB
TPITCH = 131
EBK = EMB // 8


def _body(idx_hbm, tok_hbm, pos_hbm, out_hbm, idx_v, rows_v, pos_v, t_v,
          isem, gsem, osem):
    wid = lax.axis_index("s") * NC + lax.axis_index("c")
    pltpu.sync_copy(pos_hbm, pos_v)
    lane = lax.iota(jnp.int32, 16)
    evecs = [16 * j + lane for j in range(EMB // 16)]

    def fire_idx(pb):
        s = lax.rem(pb, 2)
        pltpu.async_copy(idx_hbm.at[pb, wid], idx_v.at[s], isem.at[s])

    def wait_idx(pb):
        s = lax.rem(pb, 2)
        pltpu.make_async_copy(idx_hbm.at[0, wid], idx_v.at[s], isem.at[s]).wait()

    def fire_gathers(c):
        s = lax.rem(c, 2)
        si = lax.rem(lax.div(c, 2), 2)
        for pr4 in range(HP):
            pltpu.async_copy(
                tok_hbm.at[idx_v.at[si, s * HP + pr4]],
                rows_v.at[pl.ds(s * HROWS + pr4 * BB, BB)],
                gsem.at[s],
            )

    def wait_gathers(c):
        s = lax.rem(c, 2)
        pltpu.make_async_copy(
            tok_hbm.at[pl.ds(0, HROWS)], rows_v.at[pl.ds(0, HROWS)], gsem.at[s]
        ).wait()

    def wait_store_grp(grp):
        for _ in range(2 * EBK):
            pltpu.make_async_copy(
                t_v.at[grp, 0, pl.ds(0, 8), pl.ds(0, BB)],
                out_hbm.at[0, 0, wid],
                osem.at[grp],
            ).wait()

    def transpose_half(c, skip_wait=False):
        s = lax.rem(c, 2)
        rbase = s * HROWS
        for grp in range(2):
            if not skip_wait:
                wait_store_grp(grp)
            for p4i in range(2):
                p4 = grp * 2 + p4i
                p_abs = c * HP + p4
                tref = t_v.at[grp, p4i]
                pvs = [pos_v[p_abs, pl.ds(16 * j, 16)] for j in range(EMB // 16)]

                @plsc.parallel_loop(0, BB, step=1)
                def rbody(r):
                    bc = jnp.full((16,), r, dtype=jnp.int32)
                    row = rbase + p4 * BB + r
                    for j in range(EMB // 16):
                        vals = rows_v[row, pl.ds(16 * j, 16)] + pvs[j]
                        plsc.store_scatter(tref, [evecs[j], bc], vals)
            for p4i in range(2):
                p4 = grp * 2 + p4i
                p_abs = c * HP + p4
                for ebk in range(EBK):
                    pltpu.async_copy(
                        t_v.at[grp, p4i, pl.ds(ebk * 8, 8), pl.ds(0, BB)],
                        out_hbm.at[p_abs, ebk, wid],
                        osem.at[grp],
                    )

    fire_idx(0)
    fire_idx(1)
    wait_idx(0)
    fire_gathers(0)
    fire_gathers(1)
    wait_gathers(0)
    transpose_half(0, skip_wait=True)
    wait_idx(1)
    fire_gathers(2)
    wait_gathers(1)
    fire_idx(2)
    transpose_half(1)
    fire_gathers(3)
    wait_gathers(2)

    def step(pb, carry):
        transpose_half(2 * pb)
        wait_idx(pb + 1)
        fire_gathers(2 * pb + 2)
        wait_gathers(2 * pb + 1)
        fire_idx(pb + 2)
        transpose_half(2 * pb + 1)
        fire_gathers(2 * pb + 3)
        wait_gathers(2 * pb + 2)
        return carry

    lax.fori_loop(1, NCHUNK - 2, step, 0)

    transpose_half(46)
    wait_idx(24)
    fire_gathers(48)
    wait_gathers(47)
    transpose_half(47)
    fire_gathers(49)
    wait_gathers(48)
    transpose_half(48)
    wait_gathers(49)
    transpose_half(49)
    wait_store_grp(0)
    wait_store_grp(1)


@jax.jit
def kernel(input, tok_table, pos_table):
    idx_r = input.T.reshape(NCHUNK, PB, NW, BB).transpose(0, 2, 1, 3)
    mesh = plsc.VectorSubcoreMesh(core_axis_name="c", subcore_axis_name="s")
    out5d = pl.kernel(
        _body,
        out_type=jax.ShapeDtypeStruct((BLOCK, EBK, NW, 8, BB), jnp.float32),
        mesh=mesh,
        scratch_types=[
            pltpu.VMEM((2, PB, BB), jnp.int32),
            pltpu.VMEM((2 * HROWS, EMB), jnp.float32),
            pltpu.VMEM((BLOCK, EMB), jnp.float32),
            pltpu.VMEM((2, 2, EMB, TPITCH), jnp.float32),
            pltpu.SemaphoreType.DMA((2,)),
            pltpu.SemaphoreType.DMA((2,)),
            pltpu.SemaphoreType.DMA((2,)),
        ],
        compiler_params=pltpu.CompilerParams(
            use_tc_tiling_on_sc=False, needs_layout_passes=False
        ),
    )(idx_r, tok_table, pos_table)
    out = out5d.transpose(2, 4, 0, 1, 3).reshape(BATCH, BLOCK, EMB)
    return out


# speedup vs baseline: 2.2137x; 1.0058x over previous
import jax
import jax.numpy as jnp
from jax import lax
from jax.experimental import pallas as pl
from jax.experimental.pallas import tpu as pltpu
from jax.experimental.pallas import tpu_sc as plsc

BATCH = 4096
BLOCK = 200
EMB = 64
NC = 2
NS = 16
NW = NC * NS
BB = BATCH // NW
PB = 8
NCHUNK = BLOCK // PB
HP = 4
HROWS = HP * BB
TPITCH = 131
EBK = EMB // 8


def _body(idx_hbm, tok_hbm, pos_hbm, out_hbm, idx_v, rows_v, pos_v, t_v,
          isem, gsem, osem):
    wid = lax.axis_index("s") * NC + lax.axis_index("c")
    pltpu.sync_copy(pos_hbm, pos_v)
    lane = lax.iota(jnp.int32, 16)
    evecs = [16 * j + lane for j in range(EMB // 16)]

    def fire_idx(pb):
        s = lax.rem(pb, 2)
        pltpu.async_copy(idx_hbm.at[pb, wid], idx_v.at[s], isem.at[s])

    def wait_idx(pb):
        s = lax.rem(pb, 2)
        pltpu.make_async_copy(idx_hbm.at[0, wid], idx_v.at[s], isem.at[s]).wait()

    def fire_gathers(c):
        s = lax.rem(c, 2)
        si = lax.rem(lax.div(c, 2), 2)
        for pr4 in range(HP):
            pltpu.async_copy(
                tok_hbm.at[idx_v.at[si, s * HP + pr4]],
                rows_v.at[pl.ds(s * HROWS + pr4 * BB, BB)],
                gsem.at[s],
            )

    def wait_gathers(c):
        s = lax.rem(c, 2)
        pltpu.make_async_copy(
            tok_hbm.at[pl.ds(0, HROWS)], rows_v.at[pl.ds(0, HROWS)], gsem.at[s]
        ).wait()

    def wait_store_grp(grp):
        for _ in range(2 * EBK):
            pltpu.make_async_copy(
                t_v.at[grp, 0, pl.ds(0, 8), pl.ds(0, BB)],
                out_hbm.at[0, 0, wid],
                osem.at[grp],
            ).wait()

    def transpose_half(c, skip_wait=False):
        s = lax.rem(c, 2)
        rbase = s * HROWS
        for grp in range(2):
            if not skip_wait:
                wait_store_grp(grp)
            for p4i in range(2):
                p4 = grp * 2 + p4i
                p_abs = c * HP + p4
                tref = t_v.at[grp, p4i]
                pvs = [pos_v[p_abs, pl.ds(16 * j, 16)] for j in range(EMB // 16)]

                @plsc.parallel_loop(0, BB, step=1, unroll=2)
                def rbody(r):
                    bc = jnp.full((16,), r, dtype=jnp.int32)
                    row = rbase + p4 * BB + r
                    for j in range(EMB // 16):
                        vals = rows_v[row, pl.ds(16 * j, 16)] + pvs[j]
                        plsc.store_scatter(tref, [evecs[j], bc], vals)
            for p4i in range(2):
                p4 = grp * 2 + p4i
                p_abs = c * HP + p4
                for ebk in range(EBK):
                    pltpu.async_copy(
                        t_v.at[grp, p4i, pl.ds(ebk * 8, 8), pl.ds(0, BB)],
                        out_hbm.at[p_abs, ebk, wid],
                        osem.at[grp],
                    )

    fire_idx(0)
    fire_idx(1)
    wait_idx(0)
    fire_gathers(0)
    fire_gathers(1)
    wait_gathers(0)
    transpose_half(0, skip_wait=True)
    wait_idx(1)
    fire_gathers(2)
    wait_gathers(1)
    fire_idx(2)
    transpose_half(1)
    fire_gathers(3)
    wait_gathers(2)

    def step(pb, carry):
        transpose_half(2 * pb)
        wait_idx(pb + 1)
        fire_gathers(2 * pb + 2)
        wait_gathers(2 * pb + 1)
        fire_idx(pb + 2)
        transpose_half(2 * pb + 1)
        fire_gathers(2 * pb + 3)
        wait_gathers(2 * pb + 2)
        return carry

    lax.fori_loop(1, NCHUNK - 2, step, 0)

    transpose_half(46)
    wait_idx(24)
    fire_gathers(48)
    wait_gathers(47)
    transpose_half(47)
    fire_gathers(49)
    wait_gathers(48)
    transpose_half(48)
    wait_gathers(49)
    transpose_half(49)
    wait_store_grp(0)
    wait_store_grp(1)


@jax.jit
def kernel(input, tok_table, pos_table):
    idx_r = input.T.reshape(NCHUNK, PB, NW, BB).transpose(0, 2, 1, 3)
    mesh = plsc.VectorSubcoreMesh(core_axis_name="c", subcore_axis_name="s")
    out5d = pl.kernel(
        _body,
        out_type=jax.ShapeDtypeStruct((BLOCK, EBK, NW, 8, BB), jnp.float32),
        mesh=mesh,
        scratch_types=[
            pltpu.VMEM((2, PB, BB), jnp.int32),
            pltpu.VMEM((2 * HROWS, EMB), jnp.float32),
            pltpu.VMEM((BLOCK, EMB), jnp.float32),
            pltpu.VMEM((2, 2, EMB, TPITCH), jnp.float32),
            pltpu.SemaphoreType.DMA((2,)),
            pltpu.SemaphoreType.DMA((2,)),
            pltpu.SemaphoreType.DMA((2,)),
        ],
        compiler_params=pltpu.CompilerParams(
            use_tc_tiling_on_sc=False, needs_layout_passes=False
        ),
    )(idx_r, tok_table, pos_table)
    out = out5d.transpose(2, 4, 0, 1, 3).reshape(BATCH, BLOCK, EMB)
    return out
